# baseline (speedup 1.0000x reference)
import functools

import jax
import jax.numpy as jnp
from jax import lax
from jax.experimental import pallas as pl
from jax.experimental.pallas import tpu as pltpu
from jax.experimental.pallas import tpu_sc as plsc

N, E, D, H, C = 10000, 320000, 128, 96, 40
HP = 128
NC, NS = 2, 16
LANES = 16
CHUNK = 104
NCHUNK = 100
EPAD = NC * NS * NCHUNK * CHUNK
NIX = 4
NRB = 2
RPT = 640
NPAD = NS * RPT


@functools.partial(
    pl.kernel,
    out_type=jax.ShapeDtypeStruct((NC, NPAD, HP), jnp.float32),
    mesh=plsc.VectorSubcoreMesh(core_axis_name="c", subcore_axis_name="s"),
    compiler_params=pltpu.CompilerParams(needs_layout_passes=False),
    scratch_types=[
        pltpu.VMEM((NIX, CHUNK), jnp.int32),
        pltpu.VMEM((NIX, CHUNK), jnp.int32),
        pltpu.VMEM((CHUNK, HP), jnp.float32),
        pltpu.VMEM((CHUNK, HP), jnp.float32),
        pltpu.VMEM((8, HP), jnp.float32),
        pltpu.VMEM_SHARED((NPAD, HP), jnp.float32),
        pltpu.SemaphoreType.DMA,
        pltpu.SemaphoreType.DMA,
        pltpu.SemaphoreType.DMA,
        pltpu.SemaphoreType.DMA,
        pltpu.SemaphoreType.DMA,
        pltpu.SemaphoreType.DMA,
        pltpu.SemaphoreType.DMA,
        pltpu.SemaphoreType.DMA,
    ],
)
def _agg(h_hbm, src_hbm, dst_hbm, out_hbm, ixsrc_v, ixdst_v, rows0_v,
         rows1_v, zbuf_v, agg_s, isem0, isem1, isem2, isem3, gsem0, gsem1,
         ssem0, ssem1):
    c = lax.axis_index("c")
    s = lax.axis_index("s")

    zeros = jnp.zeros((LANES,), jnp.float32)
    for i in range(8):
        for j in range(HP // LANES):
            zbuf_v[i, pl.ds(j * LANES, LANES)] = zeros

    def zslab(k, carry):
        pltpu.sync_copy(zbuf_v, agg_s.at[pl.ds(s * RPT + k * 8, 8)])
        return carry

    lax.fori_loop(0, RPT // 8, zslab, 0)

    rbs = (rows0_v, rows1_v)
    isems = (isem0, isem1, isem2, isem3)
    gsems = (gsem0, gsem1)
    ssems = (ssem0, ssem1)

    def ix_start(j, k):
        pltpu.async_copy(src_hbm.at[c, s, j], ixsrc_v.at[k], isems[k])
        pltpu.async_copy(dst_hbm.at[c, s, j], ixdst_v.at[k], isems[k])

    def ix_wait(j, k):
        pltpu.make_async_copy(src_hbm.at[c, s, j], ixsrc_v.at[k],
                              isems[k]).wait()
        pltpu.make_async_copy(dst_hbm.at[c, s, j], ixdst_v.at[k],
                              isems[k]).wait()

    def g_start(k, b):
        pltpu.async_copy(h_hbm.at[ixsrc_v.at[k]], rbs[b], gsems[b])

    def g_wait(k, b):
        pltpu.make_async_copy(h_hbm.at[ixsrc_v.at[k]], rbs[b],
                              gsems[b]).wait()

    def s_start(k, b):
        pltpu.async_copy(rbs[b], agg_s.at[ixdst_v.at[k]], ssems[b],
                         add=True)

    def s_wait(k, b):
        pltpu.make_async_copy(rbs[b], agg_s.at[ixdst_v.at[k]],
                              ssems[b]).wait()

    plsc.subcore_barrier()

    ix_start(0, 0)
    ix_start(1, 1)
    ix_start(2, 2)
    ix_wait(0, 0)
    g_start(0, 0)
    g_wait(0, 0)
    ix_wait(1, 1)
    g_start(1, 1)
    s_start(0, 0)
    ix_start(3, 3)
    g_wait(1, 1)
    s_wait(0, 0)
    ix_wait(2, 2)
    g_start(2, 0)
    s_start(1, 1)
    ix_start(4, 0)

    def steady(t, carry):
        for u in range(NIX):
            j = NIX * t + 2 + u
            k = (2 + u) % NIX
            b = u % NRB
            g_wait(k, b)
            s_wait((1 + u) % NIX, 1 - b)
            ix_wait(j + 1, (3 + u) % NIX)
            g_start((3 + u) % NIX, 1 - b)
            s_start(k, b)

            @pl.when(j + 3 < NCHUNK)
            def _():
                ix_start(j + 3, (1 + u) % NIX)
        return carry

    lax.fori_loop(0, (NCHUNK - 4) // NIX, steady, 0)

    g_wait(2, 0)
    s_wait(1, 1)
    ix_wait(NCHUNK - 1, 3)
    g_start(3, 1)
    s_start(2, 0)
    g_wait(3, 1)
    s_wait(2, 0)
    s_start(3, 1)
    s_wait(3, 1)
    plsc.subcore_barrier()

    pltpu.sync_copy(
        agg_s.at[pl.ds(s * RPT, RPT)],
        out_hbm.at[c, pl.ds(s * RPT, RPT)])


def _zpad(u):
    return jnp.concatenate(
        [u, jnp.zeros((N, HP - H), jnp.float32)], axis=1)


def _agg_sum(h_ref, p_ref):
    return h_ref[:, :H] + p_ref[0, :N, :H] + p_ref[1, :N, :H]


def _emb_body(x_ref, w_ref, b_ref, o_ref):
    u = (jnp.dot(x_ref[...], w_ref[...], preferred_element_type=jnp.float32)
         + b_ref[...])
    o_ref[...] = _zpad(u)


def _bn_relu(u, g, be):
    mean = jnp.mean(u, axis=0, keepdims=True)
    var = jnp.mean((u - mean) ** 2, axis=0, keepdims=True)
    return jnp.maximum(g * (u - mean) * lax.rsqrt(var + 1e-5) + be, 0.0)


def _gin_body(h_ref, p_ref, w_ref, b_ref, g_ref, be_ref, o_ref):
    t = _agg_sum(h_ref, p_ref)
    u = (jnp.dot(t, w_ref[...], preferred_element_type=jnp.float32)
         + b_ref[...])
    o_ref[...] = _zpad(_bn_relu(u, g_ref[...], be_ref[...]))


def _fin_body(h_ref, p_ref, w_ref, b_ref, g_ref, be_ref, wro_ref, bro_ref,
              o_ref):
    t = _agg_sum(h_ref, p_ref)
    u = (jnp.dot(t, w_ref[...], preferred_element_type=jnp.float32)
         + b_ref[...])
    h2 = _bn_relu(u, g_ref[...], be_ref[...])
    z = (jnp.dot(h2, wro_ref[...], preferred_element_type=jnp.float32)
         + bro_ref[...])
    z = z - jnp.max(z, axis=1, keepdims=True)
    o_ref[...] = z - jnp.log(jnp.sum(jnp.exp(z), axis=1, keepdims=True))


_emb = pl.pallas_call(
    _emb_body, out_shape=jax.ShapeDtypeStruct((N, HP), jnp.float32))
_gin = pl.pallas_call(
    _gin_body, out_shape=jax.ShapeDtypeStruct((N, HP), jnp.float32))
_fin = pl.pallas_call(
    _fin_body, out_shape=jax.ShapeDtypeStruct((N, C), jnp.float32))


def kernel(x, edge_index, W_emb, b_emb, W1, b1, g1, be1, W2, b2, g2, be2,
           W_ro, b_ro):
    npad_e = EPAD - E
    src = jnp.concatenate(
        [edge_index[0], jnp.zeros((npad_e,), jnp.int32)])
    dst = jnp.concatenate(
        [edge_index[1],
         N + jnp.arange(npad_e, dtype=jnp.int32) % (NPAD - N)])
    src4 = src.reshape(NC, NS, NCHUNK, CHUNK)
    dst4 = dst.reshape(NC, NS, NCHUNK, CHUNK)
    h0 = _emb(x, W_emb, b_emb.reshape(1, H))
    p = _agg(h0, src4, dst4)
    h1 = _gin(h0, p, W1, b1.reshape(1, H), g1.reshape(1, H),
              be1.reshape(1, H))
    p = _agg(h1, src4, dst4)
    return _fin(h1, p, W2, b2.reshape(1, H), g2.reshape(1, H),
                be2.reshape(1, H), W_ro, b_ro.reshape(1, C))


# speedup vs baseline: 3.9095x; 3.9095x over previous
import functools

import jax
import jax.numpy as jnp
from jax import lax
from jax.experimental import pallas as pl
from jax.experimental.pallas import tpu as pltpu
from jax.experimental.pallas import tpu_sc as plsc

N, E, D, H, C = 10000, 320000, 128, 96, 40
HP = 128
NC, NS = 2, 16
LANES = 16
CHUNK = 125
NCHUNK = 80
EPAD = NC * NS * NCHUNK * CHUNK
RPT = 640
NPAD = NS * RPT


@functools.partial(
    pl.kernel,
    out_type=jax.ShapeDtypeStruct((NC, NPAD, HP), jnp.float32),
    mesh=plsc.VectorSubcoreMesh(core_axis_name="c", subcore_axis_name="s"),
    compiler_params=pltpu.CompilerParams(needs_layout_passes=False),
    scratch_types=[
        pltpu.VMEM((NCHUNK, CHUNK), jnp.int32),
        pltpu.VMEM((NCHUNK, CHUNK), jnp.int32),
        pltpu.VMEM((CHUNK, HP), jnp.float32),
        pltpu.VMEM((8, HP), jnp.float32),
        pltpu.VMEM_SHARED((NPAD, HP), jnp.float32),
        pltpu.SemaphoreType.DMA,
    ],
)
def _agg(h_hbm, src_hbm, dst_hbm, out_hbm, src_v, dst_v, rows_v,
         zbuf_v, agg_s, gsem):
    c = lax.axis_index("c")
    s = lax.axis_index("s")

    pltpu.sync_copy(src_hbm.at[c, s], src_v)
    pltpu.sync_copy(dst_hbm.at[c, s], dst_v)

    zeros = jnp.zeros((LANES,), jnp.float32)
    for i in range(8):
        for j in range(HP // LANES):
            zbuf_v[i, pl.ds(j * LANES, LANES)] = zeros

    def zslab(k, carry):
        pltpu.sync_copy(zbuf_v, agg_s.at[pl.ds(s * RPT + k * 8, 8)])
        return carry

    lax.fori_loop(0, RPT // 8, zslab, 0)
    plsc.subcore_barrier()

    def edge_chunk(j, carry):
        pltpu.async_copy(h_hbm.at[src_v.at[j]], rows_v, gsem).wait()
        pltpu.sync_copy(rows_v, agg_s.at[dst_v.at[j]], add=True)
        return carry

    lax.fori_loop(0, NCHUNK, edge_chunk, 0)
    plsc.subcore_barrier()

    pltpu.sync_copy(
        agg_s.at[pl.ds(s * RPT, RPT)],
        out_hbm.at[c, pl.ds(s * RPT, RPT)])


def _zpad(u):
    return jnp.concatenate(
        [u, jnp.zeros((N, HP - H), jnp.float32)], axis=1)


def _agg_sum(h_ref, p_ref):
    return h_ref[:, :H] + p_ref[0, :N, :H] + p_ref[1, :N, :H]


def _emb_body(x_ref, w_ref, b_ref, o_ref):
    u = (jnp.dot(x_ref[...], w_ref[...], preferred_element_type=jnp.float32)
         + b_ref[...])
    o_ref[...] = _zpad(u)


def _bn_relu(u, g, be):
    mean = jnp.mean(u, axis=0, keepdims=True)
    var = jnp.mean((u - mean) ** 2, axis=0, keepdims=True)
    return jnp.maximum(g * (u - mean) * lax.rsqrt(var + 1e-5) + be, 0.0)


def _gin_body(h_ref, p_ref, w_ref, b_ref, g_ref, be_ref, o_ref):
    t = _agg_sum(h_ref, p_ref)
    u = (jnp.dot(t, w_ref[...], preferred_element_type=jnp.float32)
         + b_ref[...])
    o_ref[...] = _zpad(_bn_relu(u, g_ref[...], be_ref[...]))


def _fin_body(h_ref, p_ref, w_ref, b_ref, g_ref, be_ref, wro_ref, bro_ref,
              o_ref):
    t = _agg_sum(h_ref, p_ref)
    u = (jnp.dot(t, w_ref[...], preferred_element_type=jnp.float32)
         + b_ref[...])
    h2 = _bn_relu(u, g_ref[...], be_ref[...])
    z = (jnp.dot(h2, wro_ref[...], preferred_element_type=jnp.float32)
         + bro_ref[...])
    z = z - jnp.max(z, axis=1, keepdims=True)
    o_ref[...] = z - jnp.log(jnp.sum(jnp.exp(z), axis=1, keepdims=True))


_emb = pl.pallas_call(
    _emb_body, out_shape=jax.ShapeDtypeStruct((N, HP), jnp.float32))
_gin = pl.pallas_call(
    _gin_body, out_shape=jax.ShapeDtypeStruct((N, HP), jnp.float32))
_fin = pl.pallas_call(
    _fin_body, out_shape=jax.ShapeDtypeStruct((N, C), jnp.float32))


def kernel(x, edge_index, W_emb, b_emb, W1, b1, g1, be1, W2, b2, g2, be2,
           W_ro, b_ro):
    npad_e = EPAD - E
    src = jnp.concatenate(
        [edge_index[0], jnp.zeros((npad_e,), jnp.int32)])
    dst = jnp.concatenate(
        [edge_index[1],
         N + jnp.arange(npad_e, dtype=jnp.int32) % (NPAD - N)])
    src4 = src.reshape(NC, NS, NCHUNK, CHUNK)
    dst4 = dst.reshape(NC, NS, NCHUNK, CHUNK)
    h0 = _emb(x, W_emb, b_emb.reshape(1, H))
    p = _agg(h0, src4, dst4)
    h1 = _gin(h0, p, W1, b1.reshape(1, H), g1.reshape(1, H),
              be1.reshape(1, H))
    p = _agg(h1, src4, dst4)
    return _fin(h1, p, W2, b2.reshape(1, H), g2.reshape(1, H),
                be2.reshape(1, H), W_ro, b_ro.reshape(1, C))


# speedup vs baseline: 4.1789x; 1.0689x over previous
import functools

import jax
import jax.numpy as jnp
from jax import lax
from jax.experimental import pallas as pl
from jax.experimental.pallas import tpu as pltpu
from jax.experimental.pallas import tpu_sc as plsc

N, E, D, H, C = 10000, 320000, 128, 96, 40
HP = 128
NC, NS = 2, 16
LANES = 16
CHUNK = 125
NCHUNK = 80
EPAD = NC * NS * NCHUNK * CHUNK
RPT = 640
NPAD = NS * RPT


@functools.partial(
    pl.kernel,
    out_type=jax.ShapeDtypeStruct((NC, NPAD, H), jnp.float32),
    mesh=plsc.VectorSubcoreMesh(core_axis_name="c", subcore_axis_name="s"),
    compiler_params=pltpu.CompilerParams(
        needs_layout_passes=False, use_tc_tiling_on_sc=False),
    scratch_types=[
        pltpu.VMEM((NCHUNK, CHUNK), jnp.int32),
        pltpu.VMEM((NCHUNK, CHUNK), jnp.int32),
        pltpu.VMEM((CHUNK, H), jnp.float32),
        pltpu.VMEM((8, H), jnp.float32),
        pltpu.VMEM_SHARED((NPAD, H), jnp.float32),
        pltpu.SemaphoreType.DMA,
    ],
)
def _agg(h_hbm, src_hbm, dst_hbm, out_hbm, src_v, dst_v, rows_v,
         zbuf_v, agg_s, gsem):
    c = lax.axis_index("c")
    s = lax.axis_index("s")

    pltpu.sync_copy(src_hbm.at[c, s], src_v)
    pltpu.sync_copy(dst_hbm.at[c, s], dst_v)

    zeros = jnp.zeros((LANES,), jnp.float32)
    for i in range(8):
        for j in range(H // LANES):
            zbuf_v[i, pl.ds(j * LANES, LANES)] = zeros

    def zslab(k, carry):
        pltpu.sync_copy(zbuf_v, agg_s.at[pl.ds(s * RPT + k * 8, 8)])
        return carry

    lax.fori_loop(0, RPT // 8, zslab, 0)
    plsc.subcore_barrier()

    def edge_chunk(j, carry):
        pltpu.async_copy(h_hbm.at[src_v.at[j]], rows_v, gsem).wait()
        pltpu.sync_copy(rows_v, agg_s.at[dst_v.at[j]], add=True)
        return carry

    lax.fori_loop(0, NCHUNK, edge_chunk, 0)
    plsc.subcore_barrier()

    pltpu.sync_copy(
        agg_s.at[pl.ds(s * RPT, RPT)],
        out_hbm.at[c, pl.ds(s * RPT, RPT)])


def _agg_sum(h_ref, p_ref):
    return h_ref[...] + p_ref[0, :N] + p_ref[1, :N]


def _emb_body(x_ref, w_ref, b_ref, o_ref):
    u = (jnp.dot(x_ref[...], w_ref[...], preferred_element_type=jnp.float32)
         + b_ref[...])
    o_ref[...] = u


def _bn_relu(u, g, be):
    mean = jnp.mean(u, axis=0, keepdims=True)
    var = jnp.mean((u - mean) ** 2, axis=0, keepdims=True)
    return jnp.maximum(g * (u - mean) * lax.rsqrt(var + 1e-5) + be, 0.0)


def _gin_body(h_ref, p_ref, w_ref, b_ref, g_ref, be_ref, o_ref):
    t = _agg_sum(h_ref, p_ref)
    u = (jnp.dot(t, w_ref[...], preferred_element_type=jnp.float32)
         + b_ref[...])
    o_ref[...] = _bn_relu(u, g_ref[...], be_ref[...])


def _fin_body(h_ref, p_ref, w_ref, b_ref, g_ref, be_ref, wro_ref, bro_ref,
              o_ref):
    t = _agg_sum(h_ref, p_ref)
    u = (jnp.dot(t, w_ref[...], preferred_element_type=jnp.float32)
         + b_ref[...])
    h2 = _bn_relu(u, g_ref[...], be_ref[...])
    z = (jnp.dot(h2, wro_ref[...], preferred_element_type=jnp.float32)
         + bro_ref[...])
    z = z - jnp.max(z, axis=1, keepdims=True)
    o_ref[...] = z - jnp.log(jnp.sum(jnp.exp(z), axis=1, keepdims=True))


_emb = pl.pallas_call(
    _emb_body, out_shape=jax.ShapeDtypeStruct((N, H), jnp.float32))
_gin = pl.pallas_call(
    _gin_body, out_shape=jax.ShapeDtypeStruct((N, H), jnp.float32))
_fin = pl.pallas_call(
    _fin_body, out_shape=jax.ShapeDtypeStruct((N, C), jnp.float32))


def kernel(x, edge_index, W_emb, b_emb, W1, b1, g1, be1, W2, b2, g2, be2,
           W_ro, b_ro):
    npad_e = EPAD - E
    src = jnp.concatenate(
        [edge_index[0], jnp.zeros((npad_e,), jnp.int32)])
    dst = jnp.concatenate(
        [edge_index[1],
         N + jnp.arange(npad_e, dtype=jnp.int32) % (NPAD - N)])
    src4 = src.reshape(NC, NS, NCHUNK, CHUNK)
    dst4 = dst.reshape(NC, NS, NCHUNK, CHUNK)
    h0 = _emb(x, W_emb, b_emb.reshape(1, H))
    p = _agg(h0, src4, dst4)
    h1 = _gin(h0, p, W1, b1.reshape(1, H), g1.reshape(1, H),
              be1.reshape(1, H))
    p = _agg(h1, src4, dst4)
    return _fin(h1, p, W2, b2.reshape(1, H), g2.reshape(1, H),
                be2.reshape(1, H), W_ro, b_ro.reshape(1, C))


# speedup vs baseline: 5.1069x; 1.2221x over previous
import functools

import jax
import jax.numpy as jnp
from jax import lax
from jax.experimental import pallas as pl
from jax.experimental.pallas import tpu as pltpu
from jax.experimental.pallas import tpu_sc as plsc

N, E, D, H, C = 10000, 320000, 128, 96, 40
HP = 128
NC, NS = 2, 16
LANES = 16
CHUNK = 125
NCHUNK = 80
EPAD = NC * NS * NCHUNK * CHUNK
RPT = 640
NPAD = NS * RPT


@functools.partial(
    pl.kernel,
    out_type=jax.ShapeDtypeStruct((NC, NPAD, H), jnp.float32),
    mesh=plsc.VectorSubcoreMesh(core_axis_name="c", subcore_axis_name="s"),
    compiler_params=pltpu.CompilerParams(
        needs_layout_passes=False, use_tc_tiling_on_sc=False),
    scratch_types=[
        pltpu.VMEM((NCHUNK, CHUNK), jnp.int32),
        pltpu.VMEM((NCHUNK, CHUNK), jnp.int32),
        pltpu.VMEM((CHUNK, H), jnp.float32),
        pltpu.VMEM((CHUNK, H), jnp.float32),
        pltpu.VMEM((8, H), jnp.float32),
        pltpu.VMEM_SHARED((NPAD, H), jnp.float32),
        pltpu.SemaphoreType.DMA,
        pltpu.SemaphoreType.DMA,
        pltpu.SemaphoreType.DMA,
    ],
)
def _agg(h_hbm, src_hbm, dst_hbm, out_hbm, src_v, dst_v, rows0_v, rows1_v,
         zbuf_v, agg_s, gsem, ssem0, ssem1):
    c = lax.axis_index("c")
    s = lax.axis_index("s")

    pltpu.sync_copy(src_hbm.at[c, s], src_v)
    pltpu.sync_copy(dst_hbm.at[c, s], dst_v)

    zeros = jnp.zeros((LANES,), jnp.float32)
    for i in range(8):
        for j in range(H // LANES):
            zbuf_v[i, pl.ds(j * LANES, LANES)] = zeros

    def zslab(k, carry):
        pltpu.sync_copy(zbuf_v, agg_s.at[pl.ds(s * RPT + k * 8, 8)])
        return carry

    lax.fori_loop(0, RPT // 8, zslab, 0)
    plsc.subcore_barrier()

    rbs = (rows0_v, rows1_v)
    ssems = (ssem0, ssem1)

    def s_start(j, b):
        pltpu.async_copy(rbs[b], agg_s.at[dst_v.at[j]], ssems[b], add=True)

    def s_wait(j, b):
        pltpu.make_async_copy(rbs[b], agg_s.at[dst_v.at[j]],
                              ssems[b]).wait()

    for b in range(2):
        pltpu.async_copy(h_hbm.at[src_v.at[b]], rbs[b], gsem).wait()
        s_start(b, b)

    def edge_chunk(t, carry):
        for b in range(2):
            j = 2 * t + b
            s_wait(j - 2, b)
            pltpu.async_copy(h_hbm.at[src_v.at[j]], rbs[b], gsem).wait()
            s_start(j, b)
        return carry

    lax.fori_loop(1, NCHUNK // 2, edge_chunk, 0)
    s_wait(NCHUNK - 2, 0)
    s_wait(NCHUNK - 1, 1)
    plsc.subcore_barrier()

    pltpu.sync_copy(
        agg_s.at[pl.ds(s * RPT, RPT)],
        out_hbm.at[c, pl.ds(s * RPT, RPT)])


def _agg_sum(h_ref, p_ref):
    return h_ref[...] + p_ref[0, :N] + p_ref[1, :N]


def _emb_body(x_ref, w_ref, b_ref, o_ref):
    u = (jnp.dot(x_ref[...], w_ref[...], preferred_element_type=jnp.float32)
         + b_ref[...])
    o_ref[...] = u


def _bn_relu(u, g, be):
    mean = jnp.mean(u, axis=0, keepdims=True)
    var = jnp.mean((u - mean) ** 2, axis=0, keepdims=True)
    return jnp.maximum(g * (u - mean) * lax.rsqrt(var + 1e-5) + be, 0.0)


def _gin_body(h_ref, p_ref, w_ref, b_ref, g_ref, be_ref, o_ref):
    t = _agg_sum(h_ref, p_ref)
    u = (jnp.dot(t, w_ref[...], preferred_element_type=jnp.float32)
         + b_ref[...])
    o_ref[...] = _bn_relu(u, g_ref[...], be_ref[...])


def _fin_body(h_ref, p_ref, w_ref, b_ref, g_ref, be_ref, wro_ref, bro_ref,
              o_ref):
    t = _agg_sum(h_ref, p_ref)
    u = (jnp.dot(t, w_ref[...], preferred_element_type=jnp.float32)
         + b_ref[...])
    h2 = _bn_relu(u, g_ref[...], be_ref[...])
    z = (jnp.dot(h2, wro_ref[...], preferred_element_type=jnp.float32)
         + bro_ref[...])
    z = z - jnp.max(z, axis=1, keepdims=True)
    o_ref[...] = z - jnp.log(jnp.sum(jnp.exp(z), axis=1, keepdims=True))


_emb = pl.pallas_call(
    _emb_body, out_shape=jax.ShapeDtypeStruct((N, H), jnp.float32))
_gin = pl.pallas_call(
    _gin_body, out_shape=jax.ShapeDtypeStruct((N, H), jnp.float32))
_fin = pl.pallas_call(
    _fin_body, out_shape=jax.ShapeDtypeStruct((N, C), jnp.float32))


def kernel(x, edge_index, W_emb, b_emb, W1, b1, g1, be1, W2, b2, g2, be2,
           W_ro, b_ro):
    npad_e = EPAD - E
    src = jnp.concatenate(
        [edge_index[0], jnp.zeros((npad_e,), jnp.int32)])
    dst = jnp.concatenate(
        [edge_index[1],
         N + jnp.arange(npad_e, dtype=jnp.int32) % (NPAD - N)])
    src4 = src.reshape(NC, NS, NCHUNK, CHUNK)
    dst4 = dst.reshape(NC, NS, NCHUNK, CHUNK)
    h0 = _emb(x, W_emb, b_emb.reshape(1, H))
    p = _agg(h0, src4, dst4)
    h1 = _gin(h0, p, W1, b1.reshape(1, H), g1.reshape(1, H),
              be1.reshape(1, H))
    p = _agg(h1, src4, dst4)
    return _fin(h1, p, W2, b2.reshape(1, H), g2.reshape(1, H),
                be2.reshape(1, H), W_ro, b_ro.reshape(1, C))


# speedup vs baseline: 5.2112x; 1.0204x over previous
import functools

import jax
import jax.numpy as jnp
from jax import lax
from jax.experimental import pallas as pl
from jax.experimental.pallas import tpu as pltpu
from jax.experimental.pallas import tpu_sc as plsc

N, E, D, H, C = 10000, 320000, 128, 96, 40
HP = 128
NC, NS = 2, 16
LANES = 16
CHUNK = 125
NCHUNK = 80
EPAD = NC * NS * NCHUNK * CHUNK
RPT = 640
NPAD = NS * RPT


@functools.partial(
    pl.kernel,
    out_type=jax.ShapeDtypeStruct((NC, NPAD, H), jnp.float32),
    mesh=plsc.VectorSubcoreMesh(core_axis_name="c", subcore_axis_name="s"),
    compiler_params=pltpu.CompilerParams(
        needs_layout_passes=False, use_tc_tiling_on_sc=False),
    scratch_types=[
        pltpu.VMEM((NCHUNK, CHUNK), jnp.int32),
        pltpu.VMEM((NCHUNK, CHUNK), jnp.int32),
        pltpu.VMEM((CHUNK, H), jnp.float32),
        pltpu.VMEM((CHUNK, H), jnp.float32),
        pltpu.VMEM((64, H), jnp.float32),
        pltpu.VMEM_SHARED((NPAD, H), jnp.float32),
        pltpu.SemaphoreType.DMA,
        pltpu.SemaphoreType.DMA,
        pltpu.SemaphoreType.DMA,
    ],
)
def _agg(h_hbm, src_hbm, dst_hbm, out_hbm, src_v, dst_v, rows0_v, rows1_v,
         zbuf_v, agg_s, gsem, ssem0, ssem1):
    c = lax.axis_index("c")
    s = lax.axis_index("s")

    pltpu.sync_copy(src_hbm.at[c, s], src_v)
    pltpu.sync_copy(dst_hbm.at[c, s], dst_v)

    zeros = jnp.zeros((LANES,), jnp.float32)

    def zrow(i, carry):
        for j in range(H // LANES):
            zbuf_v[i, pl.ds(j * LANES, LANES)] = zeros
        return carry

    lax.fori_loop(0, 64, zrow, 0)

    def zslab(k, carry):
        pltpu.sync_copy(zbuf_v, agg_s.at[pl.ds(s * RPT + k * 64, 64)])
        return carry

    lax.fori_loop(0, RPT // 64, zslab, 0)
    plsc.subcore_barrier()

    rbs = (rows0_v, rows1_v)
    ssems = (ssem0, ssem1)

    def s_start(j, b):
        pltpu.async_copy(rbs[b], agg_s.at[dst_v.at[j]], ssems[b], add=True)

    def s_wait(j, b):
        pltpu.make_async_copy(rbs[b], agg_s.at[dst_v.at[j]],
                              ssems[b]).wait()

    for b in range(2):
        pltpu.async_copy(h_hbm.at[src_v.at[b]], rbs[b], gsem).wait()
        s_start(b, b)

    def edge_chunk(t, carry):
        for b in range(2):
            j = 2 * t + b
            s_wait(j - 2, b)
            pltpu.async_copy(h_hbm.at[src_v.at[j]], rbs[b], gsem).wait()
            s_start(j, b)
        return carry

    lax.fori_loop(1, NCHUNK // 2, edge_chunk, 0)
    s_wait(NCHUNK - 2, 0)
    s_wait(NCHUNK - 1, 1)
    plsc.subcore_barrier()

    pltpu.sync_copy(
        agg_s.at[pl.ds(s * RPT, RPT)],
        out_hbm.at[c, pl.ds(s * RPT, RPT)])


def _agg_sum(h_ref, p_ref):
    return h_ref[...] + p_ref[0, :N] + p_ref[1, :N]


def _emb_body(x_ref, w_ref, b_ref, o_ref):
    u = (jnp.dot(x_ref[...], w_ref[...], preferred_element_type=jnp.float32)
         + b_ref[...])
    o_ref[...] = u


def _bn_relu(u, g, be):
    mean = jnp.mean(u, axis=0, keepdims=True)
    var = jnp.mean((u - mean) ** 2, axis=0, keepdims=True)
    return jnp.maximum(g * (u - mean) * lax.rsqrt(var + 1e-5) + be, 0.0)


def _gin_body(h_ref, p_ref, w_ref, b_ref, g_ref, be_ref, o_ref):
    t = _agg_sum(h_ref, p_ref)
    u = (jnp.dot(t, w_ref[...], preferred_element_type=jnp.float32)
         + b_ref[...])
    o_ref[...] = _bn_relu(u, g_ref[...], be_ref[...])


def _fin_body(h_ref, p_ref, w_ref, b_ref, g_ref, be_ref, wro_ref, bro_ref,
              o_ref):
    t = _agg_sum(h_ref, p_ref)
    u = (jnp.dot(t, w_ref[...], preferred_element_type=jnp.float32)
         + b_ref[...])
    h2 = _bn_relu(u, g_ref[...], be_ref[...])
    z = (jnp.dot(h2, wro_ref[...], preferred_element_type=jnp.float32)
         + bro_ref[...])
    z = z - jnp.max(z, axis=1, keepdims=True)
    o_ref[...] = z - jnp.log(jnp.sum(jnp.exp(z), axis=1, keepdims=True))


_emb = pl.pallas_call(
    _emb_body, out_shape=jax.ShapeDtypeStruct((N, H), jnp.float32))
_gin = pl.pallas_call(
    _gin_body, out_shape=jax.ShapeDtypeStruct((N, H), jnp.float32))
_fin = pl.pallas_call(
    _fin_body, out_shape=jax.ShapeDtypeStruct((N, C), jnp.float32))


def kernel(x, edge_index, W_emb, b_emb, W1, b1, g1, be1, W2, b2, g2, be2,
           W_ro, b_ro):
    npad_e = EPAD - E
    src = jnp.concatenate(
        [edge_index[0], jnp.zeros((npad_e,), jnp.int32)])
    dst = jnp.concatenate(
        [edge_index[1],
         N + jnp.arange(npad_e, dtype=jnp.int32) % (NPAD - N)])
    src4 = src.reshape(NC, NS, NCHUNK, CHUNK)
    dst4 = dst.reshape(NC, NS, NCHUNK, CHUNK)
    h0 = _emb(x, W_emb, b_emb.reshape(1, H))
    p = _agg(h0, src4, dst4)
    h1 = _gin(h0, p, W1, b1.reshape(1, H), g1.reshape(1, H),
              be1.reshape(1, H))
    p = _agg(h1, src4, dst4)
    return _fin(h1, p, W2, b2.reshape(1, H), g2.reshape(1, H),
                be2.reshape(1, H), W_ro, b_ro.reshape(1, C))


# speedup vs baseline: 5.2288x; 1.0034x over previous
import functools

import jax
import jax.numpy as jnp
from jax import lax
from jax.experimental import pallas as pl
from jax.experimental.pallas import tpu as pltpu
from jax.experimental.pallas import tpu_sc as plsc

N, E, D, H, C = 10000, 320000, 128, 96, 40
HP = 128
NC, NS = 2, 16
LANES = 16
CHUNK = 125
NCHUNK = 80
EPAD = NC * NS * NCHUNK * CHUNK
RPT = 640
NPAD = NS * RPT


@functools.partial(
    pl.kernel,
    out_type=jax.ShapeDtypeStruct((NC, NPAD, H), jnp.float32),
    mesh=plsc.VectorSubcoreMesh(core_axis_name="c", subcore_axis_name="s"),
    compiler_params=pltpu.CompilerParams(
        needs_layout_passes=False, use_tc_tiling_on_sc=False),
    scratch_types=[
        pltpu.VMEM((NCHUNK, CHUNK), jnp.int32),
        pltpu.VMEM((NCHUNK, CHUNK), jnp.int32),
        pltpu.VMEM((CHUNK, H), jnp.float32),
        pltpu.VMEM((CHUNK, H), jnp.float32),
        pltpu.VMEM((CHUNK, H), jnp.float32),
        pltpu.VMEM((64, H), jnp.float32),
        pltpu.VMEM_SHARED((NPAD, H), jnp.float32),
        pltpu.SemaphoreType.DMA,
        pltpu.SemaphoreType.DMA,
        pltpu.SemaphoreType.DMA,
        pltpu.SemaphoreType.DMA,
    ],
)
def _agg(h_hbm, src_hbm, dst_hbm, out_hbm, src_v, dst_v, rows0_v, rows1_v,
         rows2_v, zbuf_v, agg_s, gsem, ssem0, ssem1, ssem2):
    c = lax.axis_index("c")
    s = lax.axis_index("s")

    pltpu.sync_copy(src_hbm.at[c, s], src_v)
    pltpu.sync_copy(dst_hbm.at[c, s], dst_v)

    zeros = jnp.zeros((LANES,), jnp.float32)

    def zrow(i, carry):
        for j in range(H // LANES):
            zbuf_v[i, pl.ds(j * LANES, LANES)] = zeros
        return carry

    lax.fori_loop(0, 64, zrow, 0)

    def zslab(k, carry):
        pltpu.sync_copy(zbuf_v, agg_s.at[pl.ds(s * RPT + k * 64, 64)])
        return carry

    lax.fori_loop(0, RPT // 64, zslab, 0)
    plsc.subcore_barrier()

    rbs = (rows0_v, rows1_v, rows2_v)
    ssems = (ssem0, ssem1, ssem2)

    def s_start(j, b):
        pltpu.async_copy(rbs[b], agg_s.at[dst_v.at[j]], ssems[b], add=True)

    def s_wait(j, b):
        pltpu.make_async_copy(rbs[b], agg_s.at[dst_v.at[j]],
                              ssems[b]).wait()

    for b in range(3):
        pltpu.async_copy(h_hbm.at[src_v.at[b]], rbs[b], gsem).wait()
        s_start(b, b)

    def edge_chunk(t, carry):
        for b in range(3):
            j = 3 * t + b
            s_wait(j - 3, b)
            pltpu.async_copy(h_hbm.at[src_v.at[j]], rbs[b], gsem).wait()
            s_start(j, b)
        return carry

    lax.fori_loop(1, (NCHUNK - 2) // 3, edge_chunk, 0)
    for j in (NCHUNK - 2, NCHUNK - 1):
        b = j % 3
        s_wait(j - 3, b)
        pltpu.async_copy(h_hbm.at[src_v.at[j]], rbs[b], gsem).wait()
        s_start(j, b)
    for j in (NCHUNK - 3, NCHUNK - 2, NCHUNK - 1):
        s_wait(j, j % 3)
    plsc.subcore_barrier()

    pltpu.sync_copy(
        agg_s.at[pl.ds(s * RPT, RPT)],
        out_hbm.at[c, pl.ds(s * RPT, RPT)])


def _agg_sum(h_ref, p_ref):
    return h_ref[...] + p_ref[0, :N] + p_ref[1, :N]


def _emb_body(x_ref, w_ref, b_ref, o_ref):
    u = (jnp.dot(x_ref[...], w_ref[...], preferred_element_type=jnp.float32)
         + b_ref[...])
    o_ref[...] = u


def _bn_relu(u, g, be):
    mean = jnp.mean(u, axis=0, keepdims=True)
    var = jnp.mean((u - mean) ** 2, axis=0, keepdims=True)
    return jnp.maximum(g * (u - mean) * lax.rsqrt(var + 1e-5) + be, 0.0)


def _gin_body(h_ref, p_ref, w_ref, b_ref, g_ref, be_ref, o_ref):
    t = _agg_sum(h_ref, p_ref)
    u = (jnp.dot(t, w_ref[...], preferred_element_type=jnp.float32)
         + b_ref[...])
    o_ref[...] = _bn_relu(u, g_ref[...], be_ref[...])


def _fin_body(h_ref, p_ref, w_ref, b_ref, g_ref, be_ref, wro_ref, bro_ref,
              o_ref):
    t = _agg_sum(h_ref, p_ref)
    u = (jnp.dot(t, w_ref[...], preferred_element_type=jnp.float32)
         + b_ref[...])
    h2 = _bn_relu(u, g_ref[...], be_ref[...])
    z = (jnp.dot(h2, wro_ref[...], preferred_element_type=jnp.float32)
         + bro_ref[...])
    z = z - jnp.max(z, axis=1, keepdims=True)
    o_ref[...] = z - jnp.log(jnp.sum(jnp.exp(z), axis=1, keepdims=True))


_emb = pl.pallas_call(
    _emb_body, out_shape=jax.ShapeDtypeStruct((N, H), jnp.float32))
_gin = pl.pallas_call(
    _gin_body, out_shape=jax.ShapeDtypeStruct((N, H), jnp.float32))
_fin = pl.pallas_call(
    _fin_body, out_shape=jax.ShapeDtypeStruct((N, C), jnp.float32))


def kernel(x, edge_index, W_emb, b_emb, W1, b1, g1, be1, W2, b2, g2, be2,
           W_ro, b_ro):
    npad_e = EPAD - E
    src = jnp.concatenate(
        [edge_index[0], jnp.zeros((npad_e,), jnp.int32)])
    dst = jnp.concatenate(
        [edge_index[1],
         N + jnp.arange(npad_e, dtype=jnp.int32) % (NPAD - N)])
    src4 = src.reshape(NC, NS, NCHUNK, CHUNK)
    dst4 = dst.reshape(NC, NS, NCHUNK, CHUNK)
    h0 = _emb(x, W_emb, b_emb.reshape(1, H))
    p = _agg(h0, src4, dst4)
    h1 = _gin(h0, p, W1, b1.reshape(1, H), g1.reshape(1, H),
              be1.reshape(1, H))
    p = _agg(h1, src4, dst4)
    return _fin(h1, p, W2, b2.reshape(1, H), g2.reshape(1, H),
                be2.reshape(1, H), W_ro, b_ro.reshape(1, C))
